# baseline (speedup 1.0000x reference)
import jax
import jax.numpy as jnp
from jax import lax
from jax.experimental import pallas as pl
from jax.experimental.pallas import tpu as pltpu
from jax.experimental.pallas import tpu_sc as plsc


N_NODES = 10000
N_MSG = 32
N_FEAT = 128
LANES = 16

NUM_CORES = 2
NUM_SUBCORES = 16
NUM_WORKERS = NUM_CORES * NUM_SUBCORES

TC_NODES = 6000
TC_BLK = 400
SC_NODES = N_NODES - TC_NODES
C_NODES = 5
ROWS = C_NODES * N_MSG
N_CHUNKS = N_NODES // C_NODES
BASE = TC_NODES // C_NODES
SC_CHUNKS = SC_NODES // C_NODES
T_PER_W = SC_CHUNKS // NUM_WORKERS


def _acc_node(buf, j, ob):
    accs = []
    for ch in range(N_FEAT // LANES):
        sl = pl.ds(ch * LANES, LANES)
        acc = buf[j * N_MSG, sl]
        for r in range(1, N_MSG):
            acc = acc + buf[j * N_MSG + r, sl]
        accs.append((sl, acc))
    for sl, acc in accs:
        ob[j, sl] = acc


def _sc_body(x_hbm, o_hbm, b0, b1, ob0, ob1, s0, s1, so0, so1):
    c = lax.axis_index("c")
    s = lax.axis_index("s")
    w = s * NUM_CORES + c

    pltpu.async_copy(x_hbm.at[BASE + w], b0, s0)
    pltpu.async_copy(x_hbm.at[BASE + w + NUM_WORKERS], b1, s1)

    def step(t, carry):
        def run(buf, ob, si, so):
            chunk = BASE + w + NUM_WORKERS * t
            @pl.when(t >= 2)
            def _():
                pltpu.make_async_copy(ob, o_hbm.at[0], so).wait()
            pltpu.make_async_copy(x_hbm.at[0], buf, si).wait()
            def node(j, carry2):
                _acc_node(buf, j, ob)
                return carry2
            lax.fori_loop(0, C_NODES, node, 0, unroll=False)
            @pl.when(t + 2 < T_PER_W)
            def _():
                pltpu.async_copy(
                    x_hbm.at[chunk + 2 * NUM_WORKERS], buf, si)
            pltpu.async_copy(ob, o_hbm.at[chunk - BASE], so)

        @pl.when(t % 2 == 0)
        def _():
            run(b0, ob0, s0, so0)

        @pl.when(t % 2 == 1)
        def _():
            run(b1, ob1, s1, so1)

        return carry

    lax.fori_loop(0, T_PER_W, step, 0, unroll=False)
    pltpu.make_async_copy(ob0, o_hbm.at[0], so0).wait()
    pltpu.make_async_copy(ob1, o_hbm.at[0], so1).wait()


def _sc_reduce(x):
    mesh = plsc.VectorSubcoreMesh(core_axis_name="c", subcore_axis_name="s")
    f = pl.kernel(
        _sc_body,
        out_type=jax.ShapeDtypeStruct((SC_CHUNKS, C_NODES, N_FEAT),
                                      jnp.float32),
        mesh=mesh,
        scratch_types=[
            pltpu.VMEM((ROWS, N_FEAT), jnp.float32),
            pltpu.VMEM((ROWS, N_FEAT), jnp.float32),
            pltpu.VMEM((C_NODES, N_FEAT), jnp.float32),
            pltpu.VMEM((C_NODES, N_FEAT), jnp.float32),
            pltpu.SemaphoreType.DMA,
            pltpu.SemaphoreType.DMA,
            pltpu.SemaphoreType.DMA,
            pltpu.SemaphoreType.DMA,
        ],
    )
    return f(x)


def _tc_reduce_body(x_ref, o_ref):
    o_ref[...] = jnp.sum(x_ref[...], axis=1)


def _tc_reduce(x):
    return pl.pallas_call(
        _tc_reduce_body,
        grid=(TC_NODES // TC_BLK,),
        in_specs=[pl.BlockSpec((TC_BLK, N_MSG, N_FEAT), lambda i: (i, 0, 0))],
        out_specs=pl.BlockSpec((TC_BLK, N_FEAT), lambda i: (i, 0)),
        out_shape=jax.ShapeDtypeStruct((N_NODES, N_FEAT), jnp.float32),
    )(x)


def kernel(messages):
    x = messages.reshape(N_NODES, N_MSG, N_FEAT)
    tc_out = _tc_reduce(x)
    sc_out = _sc_reduce(x.reshape(N_CHUNKS, ROWS, N_FEAT))
    out = lax.dynamic_update_slice(
        tc_out, sc_out.reshape(SC_NODES, N_FEAT), (TC_NODES, 0))
    return out.reshape(1, N_NODES, N_FEAT)


# speedup vs baseline: 1.4228x; 1.4228x over previous
import jax
import jax.numpy as jnp
from jax.experimental import pallas as pl


N_NODES = 10000
N_MSG = 32
N_FEAT = 128
N_BLK = 400


def _reduce_body(x_ref, o_ref):
    o_ref[...] = jnp.sum(x_ref[...], axis=1)


def kernel(messages):
    x = messages.reshape(N_NODES, N_MSG, N_FEAT)
    out = pl.pallas_call(
        _reduce_body,
        grid=(N_NODES // N_BLK,),
        in_specs=[pl.BlockSpec((N_BLK, N_MSG, N_FEAT), lambda i: (i, 0, 0))],
        out_specs=pl.BlockSpec((N_BLK, N_FEAT), lambda i: (i, 0)),
        out_shape=jax.ShapeDtypeStruct((N_NODES, N_FEAT), jnp.float32),
    )(x)
    return out.reshape(1, N_NODES, N_FEAT)
